# baseline (speedup 1.0000x reference)
import jax
import jax.numpy as jnp
from jax.experimental import pallas as pl

H = 2048
R = 512
T = 2048
KC = 256
TC = 512


def _dot(a, b):
    return jax.lax.dot_general(a, b, (((1,), (0,)), ((), ())),
                               preferred_element_type=jnp.float32)


def _mm1_kernel(x_ref, w1_ref, y_ref):
    k = pl.program_id(1)
    part = _dot(x_ref[...], w1_ref[0])

    @pl.when(k == 0)
    def _():
        y_ref[0] = part

    @pl.when(k > 0)
    def _():
        y_ref[0] = y_ref[0] + part


def _mm2_kernel(y_ref, w2_ref, h_ref):
    h_ref[0] = _dot(y_ref[0], w2_ref[0])


def _norm_kernel(h_ref, o_ref):
    t = pl.program_id(1)
    hc = h_ref[0]
    part = jnp.sum(hc * hc, axis=0, keepdims=True)

    @pl.when(t == 0)
    def _():
        o_ref[0] = part

    @pl.when(t > 0)
    def _():
        o_ref[0] = o_ref[0] + part


def _sort_kernel(qrow_ref, qcol_ref, idx_ref):
    CH = 256
    n_ch = T // CH
    qcol = qcol_ref[...]
    kcol = jax.lax.broadcasted_iota(jnp.int32, (T, 1), 0)

    def rank_body(c, rank_col):
        q_chunk = qrow_ref[:, pl.ds(c * CH, CH)]
        j_ids = jax.lax.broadcasted_iota(jnp.int32, (1, CH), 1) + c * CH
        gt = (q_chunk > qcol).astype(jnp.int32)
        eq = (q_chunk == qcol) & (j_ids < kcol)
        return rank_col + jnp.sum(gt + eq.astype(jnp.int32), axis=1,
                                  keepdims=True)

    rank_col = jax.lax.fori_loop(0, n_ch, rank_body,
                                 jnp.zeros((T, 1), jnp.int32))

    def inv_body(c, _):
        i_ids = jax.lax.broadcasted_iota(jnp.int32, (1, CH), 1) + c * CH
        onehot = (rank_col == i_ids).astype(jnp.int32)
        idx_ref[:, pl.ds(c * CH, CH)] = jnp.sum(onehot * kcol, axis=0,
                                                keepdims=True)
        return 0

    jax.lax.fori_loop(0, n_ch, inv_body, 0)


def kernel(x, sparsity, w1, w2):
    xm = x.reshape(T, H)

    y = pl.pallas_call(
        _mm1_kernel,
        grid=(2, H // KC),
        in_specs=[
            pl.BlockSpec((T, KC), lambda b, k: (0, k)),
            pl.BlockSpec((1, KC, R), lambda b, k: (b, k, 0)),
        ],
        out_specs=pl.BlockSpec((1, T, R), lambda b, k: (b, 0, 0)),
        out_shape=jax.ShapeDtypeStruct((2, T, R), jnp.float32),
    )(xm, w1)

    hm = pl.pallas_call(
        _mm2_kernel,
        grid=(2,),
        in_specs=[
            pl.BlockSpec((1, T, R), lambda b: (b, 0, 0)),
            pl.BlockSpec((1, R, H), lambda b: (b, 0, 0)),
        ],
        out_specs=pl.BlockSpec((1, T, H), lambda b: (b, 0, 0)),
        out_shape=jax.ShapeDtypeStruct((2, T, H), jnp.float32),
    )(y, w2)

    n2 = pl.pallas_call(
        _norm_kernel,
        grid=(2, T // TC),
        in_specs=[pl.BlockSpec((1, TC, H), lambda b, t: (b, t, 0))],
        out_specs=pl.BlockSpec((1, 1, H), lambda b, t: (b, 0, 0)),
        out_shape=jax.ShapeDtypeStruct((2, 1, H), jnp.float32),
    )(hm).reshape(2, H)

    n = jnp.sqrt(n2)
    qk = n[0] * n[1]
    qk = qk * (1 - sparsity)

    idx = pl.pallas_call(
        _sort_kernel,
        in_specs=[
            pl.BlockSpec((1, T), lambda: (0, 0)),
            pl.BlockSpec((T, 1), lambda: (0, 0)),
        ],
        out_specs=pl.BlockSpec((1, T), lambda: (0, 0)),
        out_shape=jax.ShapeDtypeStruct((1, T), jnp.int32),
    )(qk.reshape(1, T), qk.reshape(T, 1))
    return idx.reshape(T)


# speedup vs baseline: 1.4501x; 1.4501x over previous
import jax
import jax.numpy as jnp
from jax.experimental import pallas as pl
from jax.experimental.pallas import tpu as pltpu

H = 2048
R = 512
T = 2048
KC = 256
TC = 512


def _dot(a, b):
    return jax.lax.dot_general(a, b, (((1,), (0,)), ((), ())),
                               preferred_element_type=jnp.float32)


def _fused_kernel(x_ref, w1_ref, w2_ref, o_ref, y_sc, h_sc):
    k = pl.program_id(1)
    part = _dot(x_ref[...], w1_ref[0])

    @pl.when(k == 0)
    def _():
        y_sc[...] = part

    @pl.when(k > 0)
    def _():
        y_sc[...] = y_sc[...] + part

    @pl.when(k == H // KC - 1)
    def _():
        h_sc[...] = _dot(y_sc[...], w2_ref[0])
        acc = None
        for t in range(T // TC):
            hc = h_sc[pl.ds(t * TC, TC), :]
            p = jnp.sum(hc * hc, axis=0, keepdims=True)
            acc = p if acc is None else acc + p
        o_ref[0] = acc


def _sort_kernel(qrow_ref, qcol_ref, idx_ref):
    CH = 256
    n_ch = T // CH
    qcol = qcol_ref[...]
    kcol = jax.lax.broadcasted_iota(jnp.int32, (T, 1), 0)

    def rank_body(c, rank_col):
        q_chunk = qrow_ref[:, pl.ds(c * CH, CH)]
        j_ids = jax.lax.broadcasted_iota(jnp.int32, (1, CH), 1) + c * CH
        gt = (q_chunk > qcol).astype(jnp.int32)
        eq = (q_chunk == qcol) & (j_ids < kcol)
        return rank_col + jnp.sum(gt + eq.astype(jnp.int32), axis=1,
                                  keepdims=True)

    rank_col = jax.lax.fori_loop(0, n_ch, rank_body,
                                 jnp.zeros((T, 1), jnp.int32))

    def inv_body(c, _):
        i_ids = jax.lax.broadcasted_iota(jnp.int32, (1, CH), 1) + c * CH
        onehot = (rank_col == i_ids).astype(jnp.int32)
        idx_ref[:, pl.ds(c * CH, CH)] = jnp.sum(onehot * kcol, axis=0,
                                                keepdims=True)
        return 0

    jax.lax.fori_loop(0, n_ch, inv_body, 0)


def kernel(x, sparsity, w1, w2):
    xm = x.reshape(T, H)

    n2 = pl.pallas_call(
        _fused_kernel,
        grid=(2, H // KC),
        in_specs=[
            pl.BlockSpec((T, KC), lambda b, k: (0, k)),
            pl.BlockSpec((1, KC, R), lambda b, k: (b, k, 0)),
            pl.BlockSpec((1, R, H), lambda b, k: (b, 0, 0)),
        ],
        out_specs=pl.BlockSpec((1, 1, H), lambda b, k: (b, 0, 0)),
        out_shape=jax.ShapeDtypeStruct((2, 1, H), jnp.float32),
        scratch_shapes=[
            pltpu.VMEM((T, R), jnp.float32),
            pltpu.VMEM((T, H), jnp.float32),
        ],
    )(xm, w1, w2).reshape(2, H)

    n = jnp.sqrt(n2)
    qk = n[0] * n[1]
    qk = qk * (1 - sparsity)

    idx = pl.pallas_call(
        _sort_kernel,
        in_specs=[
            pl.BlockSpec((1, T), lambda: (0, 0)),
            pl.BlockSpec((T, 1), lambda: (0, 0)),
        ],
        out_specs=pl.BlockSpec((1, T), lambda: (0, 0)),
        out_shape=jax.ShapeDtypeStruct((1, T), jnp.int32),
    )(qk.reshape(1, T), qk.reshape(T, 1))
    return idx.reshape(T)


# speedup vs baseline: 1.6107x; 1.1108x over previous
import jax
import jax.numpy as jnp
from jax.experimental import pallas as pl
from jax.experimental.pallas import tpu as pltpu

H = 2048
R = 512
T = 2048
KC = 256
TC = 512


def _dot(a, b):
    return jax.lax.dot_general(a, b, (((1,), (0,)), ((), ())),
                               preferred_element_type=jnp.float32)


def _fused_kernel(x_ref, w1a_ref, w1b_ref, w2_ref, o_ref, y0_sc, y1_sc, h_sc):
    k = pl.program_id(0)
    part0 = _dot(x_ref[...], w1a_ref[0])
    part1 = _dot(x_ref[...], w1b_ref[0])

    @pl.when(k == 0)
    def _():
        y0_sc[...] = part0
        y1_sc[...] = part1

    @pl.when(k > 0)
    def _():
        y0_sc[...] = y0_sc[...] + part0
        y1_sc[...] = y1_sc[...] + part1

    @pl.when(k == H // KC - 1)
    def _():
        for b, y_sc in ((0, y0_sc), (1, y1_sc)):
            h_sc[...] = _dot(y_sc[...], w2_ref[b])
            acc = None
            for t in range(T // TC):
                hc = h_sc[pl.ds(t * TC, TC), :]
                p = jnp.sum(hc * hc, axis=0, keepdims=True)
                acc = p if acc is None else acc + p
            o_ref[b] = acc


def _sort_kernel(qrow_ref, qcol_ref, idx_ref):
    CH = 256
    n_ch = T // CH
    qcol = qcol_ref[...]
    kcol = jax.lax.broadcasted_iota(jnp.int32, (T, 1), 0)

    def rank_body(c, rank_col):
        q_chunk = qrow_ref[:, pl.ds(c * CH, CH)]
        j_ids = jax.lax.broadcasted_iota(jnp.int32, (1, CH), 1) + c * CH
        gt = (q_chunk > qcol).astype(jnp.int32)
        eq = (q_chunk == qcol) & (j_ids < kcol)
        return rank_col + jnp.sum(gt + eq.astype(jnp.int32), axis=1,
                                  keepdims=True)

    rank_col = jax.lax.fori_loop(0, n_ch, rank_body,
                                 jnp.zeros((T, 1), jnp.int32))

    def inv_body(c, _):
        i_ids = jax.lax.broadcasted_iota(jnp.int32, (1, CH), 1) + c * CH
        onehot = (rank_col == i_ids).astype(jnp.int32)
        idx_ref[:, pl.ds(c * CH, CH)] = jnp.sum(onehot * kcol, axis=0,
                                                keepdims=True)
        return 0

    jax.lax.fori_loop(0, n_ch, inv_body, 0)


def kernel(x, sparsity, w1, w2):
    xm = x.reshape(T, H)

    n2 = pl.pallas_call(
        _fused_kernel,
        grid=(H // KC,),
        in_specs=[
            pl.BlockSpec((T, KC), lambda k: (0, k)),
            pl.BlockSpec((1, KC, R), lambda k: (0, k, 0)),
            pl.BlockSpec((1, KC, R), lambda k: (1, k, 0)),
            pl.BlockSpec((2, R, H), lambda k: (0, 0, 0)),
        ],
        out_specs=pl.BlockSpec((2, 1, H), lambda k: (0, 0, 0)),
        out_shape=jax.ShapeDtypeStruct((2, 1, H), jnp.float32),
        scratch_shapes=[
            pltpu.VMEM((T, R), jnp.float32),
            pltpu.VMEM((T, R), jnp.float32),
            pltpu.VMEM((T, H), jnp.float32),
        ],
    )(xm, w1, w1, w2).reshape(2, H)

    n = jnp.sqrt(n2)
    qk = n[0] * n[1]
    qk = qk * (1 - sparsity)

    idx = pl.pallas_call(
        _sort_kernel,
        in_specs=[
            pl.BlockSpec((1, T), lambda: (0, 0)),
            pl.BlockSpec((T, 1), lambda: (0, 0)),
        ],
        out_specs=pl.BlockSpec((1, T), lambda: (0, 0)),
        out_shape=jax.ShapeDtypeStruct((1, T), jnp.int32),
    )(qk.reshape(1, T), qk.reshape(T, 1))
    return idx.reshape(T)


# speedup vs baseline: 1.6118x; 1.0006x over previous
import jax
import jax.numpy as jnp
from jax.experimental import pallas as pl
from jax.experimental.pallas import tpu as pltpu

H = 2048
R = 512
T = 2048
KC = 256
TC = 512


def _dot(a, b):
    return jax.lax.dot_general(a, b, (((1,), (0,)), ((), ())),
                               preferred_element_type=jnp.float32)


def _fused_kernel(x_ref, w1a_ref, w1b_ref, w2_ref, o_ref, y0_sc, y1_sc, h_sc):
    k = pl.program_id(0)
    w1cat = jnp.concatenate([w1a_ref[0], w1b_ref[0]], axis=1)
    part = _dot(x_ref[...], w1cat)
    part0 = jax.lax.slice(part, (0, 0), (T, R))
    part1 = jax.lax.slice(part, (0, R), (T, 2 * R))

    @pl.when(k == 0)
    def _():
        y0_sc[...] = part0
        y1_sc[...] = part1

    @pl.when(k > 0)
    def _():
        y0_sc[...] = y0_sc[...] + part0
        y1_sc[...] = y1_sc[...] + part1

    @pl.when(k == H // KC - 1)
    def _():
        for b, y_sc in ((0, y0_sc), (1, y1_sc)):
            h_sc[...] = _dot(y_sc[...], w2_ref[b])
            acc = None
            for t in range(T // TC):
                hc = h_sc[pl.ds(t * TC, TC), :]
                p = jnp.sum(hc * hc, axis=0, keepdims=True)
                acc = p if acc is None else acc + p
            o_ref[b] = acc


def _sort_kernel(scale_ref, nrow_ref, ncol_ref, idx_ref):
    CH = 256
    n_ch = T // CH
    scale = scale_ref[0, 0]
    n2col = ncol_ref[...]
    qcol = (jnp.sqrt(jax.lax.slice(n2col, (0, 0), (T, 1))) *
            jnp.sqrt(jax.lax.slice(n2col, (0, 1), (T, 2)))) * scale
    kcol = jax.lax.broadcasted_iota(jnp.int32, (T, 1), 0)

    def rank_body(c, rank_col):
        n2c = nrow_ref[:, pl.ds(c * CH, CH)]
        q_chunk = (jnp.sqrt(jax.lax.slice(n2c, (0, 0), (1, CH))) *
                   jnp.sqrt(jax.lax.slice(n2c, (1, 0), (2, CH)))) * scale
        j_ids = jax.lax.broadcasted_iota(jnp.int32, (1, CH), 1) + c * CH
        gt = (q_chunk > qcol).astype(jnp.int32)
        eq = (q_chunk == qcol) & (j_ids < kcol)
        return rank_col + jnp.sum(gt + eq.astype(jnp.int32), axis=1,
                                  keepdims=True)

    rank_col = jax.lax.fori_loop(0, n_ch, rank_body,
                                 jnp.zeros((T, 1), jnp.int32))

    def inv_body(c, _):
        i_ids = jax.lax.broadcasted_iota(jnp.int32, (1, CH), 1) + c * CH
        onehot = (rank_col == i_ids).astype(jnp.int32)
        idx_ref[:, pl.ds(c * CH, CH)] = jnp.sum(onehot * kcol, axis=0,
                                                keepdims=True)
        return 0

    jax.lax.fori_loop(0, n_ch, inv_body, 0)


def kernel(x, sparsity, w1, w2):
    xm = x.reshape(T, H)

    n2 = pl.pallas_call(
        _fused_kernel,
        grid=(H // KC,),
        in_specs=[
            pl.BlockSpec((T, KC), lambda k: (0, k)),
            pl.BlockSpec((1, KC, R), lambda k: (0, k, 0)),
            pl.BlockSpec((1, KC, R), lambda k: (1, k, 0)),
            pl.BlockSpec((2, R, H), lambda k: (0, 0, 0)),
        ],
        out_specs=pl.BlockSpec((2, 1, H), lambda k: (0, 0, 0)),
        out_shape=jax.ShapeDtypeStruct((2, 1, H), jnp.float32),
        scratch_shapes=[
            pltpu.VMEM((T, R), jnp.float32),
            pltpu.VMEM((T, R), jnp.float32),
            pltpu.VMEM((T, H), jnp.float32),
        ],
    )(xm, w1, w1, w2).reshape(2, H)

    scale = jnp.asarray(1 - sparsity, jnp.float32).reshape(1, 1)
    idx = pl.pallas_call(
        _sort_kernel,
        in_specs=[
            pl.BlockSpec(memory_space=pltpu.SMEM),
            pl.BlockSpec((2, T), lambda: (0, 0)),
            pl.BlockSpec((T, 2), lambda: (0, 0)),
        ],
        out_specs=pl.BlockSpec((1, T), lambda: (0, 0)),
        out_shape=jax.ShapeDtypeStruct((1, T), jnp.int32),
    )(scale, n2, n2.T)
    return idx.reshape(T)


# speedup vs baseline: 1.6128x; 1.0006x over previous
import jax
import jax.numpy as jnp
from jax.experimental import pallas as pl
from jax.experimental.pallas import tpu as pltpu

H = 2048
R = 512
T = 2048
KC = 256
TC = 512


def _dot(a, b):
    return jax.lax.dot_general(a, b, (((1,), (0,)), ((), ())),
                               preferred_element_type=jnp.float32)


def _fused_kernel(x_ref, w1a_ref, w1b_ref, w2_ref, o_ref, y0_sc, y1_sc):
    k = pl.program_id(0)
    w1cat = jnp.concatenate([w1a_ref[0], w1b_ref[0]], axis=1)
    part = _dot(x_ref[...], w1cat)
    part0 = jax.lax.slice(part, (0, 0), (T, R))
    part1 = jax.lax.slice(part, (0, R), (T, 2 * R))

    @pl.when(k == 0)
    def _():
        y0_sc[...] = part0
        y1_sc[...] = part1

    @pl.when(k > 0)
    def _():
        y0_sc[...] = y0_sc[...] + part0
        y1_sc[...] = y1_sc[...] + part1

    @pl.when(k == H // KC - 1)
    def _():
        for b, y_sc in ((0, y0_sc), (1, y1_sc)):
            acc = None
            for t in range(T // TC):
                hc = _dot(y_sc[pl.ds(t * TC, TC), :], w2_ref[b])
                p = jnp.sum(hc * hc, axis=0, keepdims=True)
                acc = p if acc is None else acc + p
            o_ref[b] = acc


def _sort_kernel(scale_ref, nrow_ref, ncol_ref, idx_ref):
    CH = 256
    n_ch = T // CH
    scale = scale_ref[0, 0]
    n2col = ncol_ref[...]
    qcol = (jnp.sqrt(jax.lax.slice(n2col, (0, 0), (T, 1))) *
            jnp.sqrt(jax.lax.slice(n2col, (0, 1), (T, 2)))) * scale
    kcol = jax.lax.broadcasted_iota(jnp.int32, (T, 1), 0)

    def rank_body(c, rank_col):
        n2c = nrow_ref[:, pl.ds(c * CH, CH)]
        q_chunk = (jnp.sqrt(jax.lax.slice(n2c, (0, 0), (1, CH))) *
                   jnp.sqrt(jax.lax.slice(n2c, (1, 0), (2, CH)))) * scale
        j_ids = jax.lax.broadcasted_iota(jnp.int32, (1, CH), 1) + c * CH
        gt = (q_chunk > qcol).astype(jnp.int32)
        eq = (q_chunk == qcol) & (j_ids < kcol)
        return rank_col + jnp.sum(gt + eq.astype(jnp.int32), axis=1,
                                  keepdims=True)

    rank_col = jax.lax.fori_loop(0, n_ch, rank_body,
                                 jnp.zeros((T, 1), jnp.int32))

    def inv_body(c, _):
        i_ids = jax.lax.broadcasted_iota(jnp.int32, (1, CH), 1) + c * CH
        onehot = (rank_col == i_ids).astype(jnp.int32)
        idx_ref[:, pl.ds(c * CH, CH)] = jnp.sum(onehot * kcol, axis=0,
                                                keepdims=True)
        return 0

    jax.lax.fori_loop(0, n_ch, inv_body, 0)


def kernel(x, sparsity, w1, w2):
    xm = x.reshape(T, H)

    n2 = pl.pallas_call(
        _fused_kernel,
        grid=(H // KC,),
        in_specs=[
            pl.BlockSpec((T, KC), lambda k: (0, k)),
            pl.BlockSpec((1, KC, R), lambda k: (0, k, 0)),
            pl.BlockSpec((1, KC, R), lambda k: (1, k, 0)),
            pl.BlockSpec((2, R, H), lambda k: (0, 0, 0)),
        ],
        out_specs=pl.BlockSpec((2, 1, H), lambda k: (0, 0, 0)),
        out_shape=jax.ShapeDtypeStruct((2, 1, H), jnp.float32),
        scratch_shapes=[
            pltpu.VMEM((T, R), jnp.float32),
            pltpu.VMEM((T, R), jnp.float32),
        ],
    )(xm, w1, w1, w2).reshape(2, H)

    scale = jnp.asarray(1 - sparsity, jnp.float32).reshape(1, 1)
    idx = pl.pallas_call(
        _sort_kernel,
        in_specs=[
            pl.BlockSpec(memory_space=pltpu.SMEM),
            pl.BlockSpec((2, T), lambda: (0, 0)),
            pl.BlockSpec((T, 2), lambda: (0, 0)),
        ],
        out_specs=pl.BlockSpec((1, T), lambda: (0, 0)),
        out_shape=jax.ShapeDtypeStruct((1, T), jnp.int32),
    )(scale, n2, n2.T)
    return idx.reshape(T)
